# Initial kernel scaffold; baseline (speedup 1.0000x reference)
import jax
import jax.numpy as jnp
from jax.experimental import pallas as pl


def kernel(context_ids, doc_ids, target_noise_ids, D, W, O):
    raise NotImplementedError("write your pallas kernel here")


# speedup vs baseline: 3.4011x; 3.4011x over previous
import functools

import jax
import jax.numpy as jnp
from jax import lax
from jax.experimental import pallas as pl
from jax.experimental.pallas import tpu as pltpu
from jax.experimental.pallas import tpu_sc as plsc

VEC = 64
CTX = 20
NOISE = 26
BATCH = 16384
LANES = 16

_info = plsc.get_sparse_core_info()
NC = _info.num_cores
NS = _info.num_subcores
NW = NC * NS
S = 32
PER_W = BATCH // NW
N_CHUNKS = PER_W // S

_mesh = plsc.VectorSubcoreMesh(core_axis_name="c", subcore_axis_name="s")


@functools.partial(
    pl.kernel,
    mesh=_mesh,
    compiler_params=pltpu.CompilerParams(
        needs_layout_passes=False, use_tc_tiling_on_sc=False),
    out_type=jax.ShapeDtypeStruct((BATCH, NOISE), jnp.float32),
    scratch_types=[
        pltpu.VMEM((S * CTX,), jnp.int32),
        pltpu.VMEM((S,), jnp.int32),
        pltpu.VMEM((896,), jnp.int32),
        pltpu.VMEM((S * CTX, VEC), jnp.float32),
        pltpu.VMEM((S, VEC), jnp.float32),
        pltpu.VMEM((896, VEC), jnp.float32),
        pltpu.VMEM((S, NOISE), jnp.float32),
        pltpu.SemaphoreType.DMA,
    ],
)
def _dm_sc(ctx_hbm, doc_hbm, tn_hbm, d_hbm, w_hbm, ot_hbm, out_hbm,
           ctx_idx, doc_idx, tn_idx, wrows, drows, otrows, obuf, sem):
    wid = lax.axis_index("s") * NC + lax.axis_index("c")
    last_lane = lax.iota(jnp.int32, LANES) == (LANES - 1)

    zeros16 = jnp.zeros((LANES,), jnp.int32)
    for t in range(S * NOISE, 896, LANES):
        tn_idx[pl.ds(t, LANES)] = zeros16

    def chunk_body(c, carry):
        b0 = pl.multiple_of(wid * PER_W + c * S, S)

        pltpu.sync_copy(ctx_hbm.at[pl.ds(b0 * CTX, S * CTX)], ctx_idx)
        pltpu.sync_copy(doc_hbm.at[pl.ds(b0, S)], doc_idx)
        pltpu.sync_copy(tn_hbm.at[pl.ds(b0 * NOISE, S * NOISE)],
                        tn_idx.at[pl.ds(0, S * NOISE)])

        cps = []
        for t in range(S * CTX // 128):
            cps.append(pltpu.async_copy(
                w_hbm.at[ctx_idx.at[pl.ds(t * 128, 128)]],
                wrows.at[pl.ds(t * 128, 128), :], sem))
        cps.append(pltpu.async_copy(d_hbm.at[doc_idx], drows, sem))
        for t in range(896 // 128):
            cps.append(pltpu.async_copy(
                ot_hbm.at[tn_idx.at[pl.ds(t * 128, 128)]],
                otrows.at[pl.ds(t * 128, 128), :], sem))
        for cp in cps:
            cp.wait()

        def row_body(b, carry2):
            x = [drows[b, pl.ds(v * LANES, LANES)] for v in range(VEC // LANES)]
            wbase = b * CTX
            for j in range(CTX):
                for v in range(VEC // LANES):
                    x[v] = x[v] + wrows[wbase + j, pl.ds(v * LANES, LANES)]
            tbase = b * NOISE
            bvec = jnp.full((LANES,), b, jnp.int32)
            for k in range(NOISE):
                p0 = x[0] * otrows[tbase + k, pl.ds(0, LANES)]
                p1 = x[1] * otrows[tbase + k, pl.ds(LANES, LANES)]
                p2 = x[2] * otrows[tbase + k, pl.ds(2 * LANES, LANES)]
                p3 = x[3] * otrows[tbase + k, pl.ds(3 * LANES, LANES)]
                cs = plsc.cumsum((p0 + p1) + (p2 + p3))
                plsc.store_scatter(
                    obuf, [bvec, jnp.full((LANES,), k, jnp.int32)], cs,
                    mask=last_lane)
            return carry2

        lax.fori_loop(0, S, row_body, 0)
        pltpu.sync_copy(obuf, out_hbm.at[pl.ds(b0, S), :])
        return carry

    lax.fori_loop(0, N_CHUNKS, chunk_body, 0)


def kernel(context_ids, doc_ids, target_noise_ids, D, W, O):
    ctx_flat = context_ids.reshape(-1).astype(jnp.int32)
    tn_flat = target_noise_ids.reshape(-1).astype(jnp.int32)
    ot = jnp.transpose(O)
    return _dm_sc(ctx_flat, doc_ids.astype(jnp.int32), tn_flat, D, W, ot)


# speedup vs baseline: 3.4048x; 1.0011x over previous
import functools

import jax
import jax.numpy as jnp
from jax import lax
from jax.experimental import pallas as pl
from jax.experimental.pallas import tpu as pltpu
from jax.experimental.pallas import tpu_sc as plsc

VEC = 64
CTX = 20
NOISE = 26
BATCH = 16384
LANES = 16

_info = plsc.get_sparse_core_info()
NC = _info.num_cores
NS = _info.num_subcores
NW = NC * NS
S = 32
PER_W = BATCH // NW
N_CHUNKS = PER_W // S
TNP = 896
CTXC = S * CTX


@functools.partial(
    pl.kernel,
    mesh=plsc.VectorSubcoreMesh(core_axis_name="c", subcore_axis_name="s"),
    compiler_params=pltpu.CompilerParams(
        needs_layout_passes=False, use_tc_tiling_on_sc=False),
    out_type=jax.ShapeDtypeStruct((BATCH, NOISE), jnp.float32),
    scratch_types=[
        pltpu.VMEM((N_CHUNKS * CTXC,), jnp.int32),
        pltpu.VMEM((S,), jnp.int32),
        pltpu.VMEM((N_CHUNKS * TNP,), jnp.int32),
        pltpu.VMEM((CTXC, VEC), jnp.float32),
        pltpu.VMEM((S, VEC), jnp.float32),
        pltpu.VMEM((TNP, VEC), jnp.float32),
        pltpu.VMEM((S, NOISE), jnp.float32),
        pltpu.SemaphoreType.DMA,
        pltpu.SemaphoreType.DMA,
    ],
)
def _dm_sc(ctx_hbm, doc_hbm, tn_hbm, d_hbm, w_hbm, ot_hbm, out_hbm,
           ctx_idx, doc_idx, tn_idx, wrows, drows, otrows, obuf, sem, osem):
    wid = lax.axis_index("s") * NC + lax.axis_index("c")
    last_lane = lax.iota(jnp.int32, LANES) == (LANES - 1)

    wbase0 = pl.multiple_of(wid * (N_CHUNKS * CTXC), N_CHUNKS * CTXC)
    tbase0 = pl.multiple_of(wid * (N_CHUNKS * TNP), N_CHUNKS * TNP)
    pltpu.sync_copy(ctx_hbm.at[pl.ds(wbase0, N_CHUNKS * CTXC)], ctx_idx)
    pltpu.sync_copy(tn_hbm.at[pl.ds(tbase0, N_CHUNKS * TNP)], tn_idx)

    def chunk_body(c, carry):
        b0 = pl.multiple_of(wid * PER_W + c * S, S)

        pltpu.sync_copy(doc_hbm.at[pl.ds(b0, S)], doc_idx)

        cps = []
        for t in range(CTXC // 128):
            cps.append(pltpu.async_copy(
                w_hbm.at[ctx_idx.at[pl.ds(c * CTXC + t * 128, 128)]],
                wrows.at[pl.ds(t * 128, 128), :], sem))
        cps.append(pltpu.async_copy(d_hbm.at[doc_idx], drows, sem))
        for t in range(TNP // 128):
            cps.append(pltpu.async_copy(
                ot_hbm.at[tn_idx.at[pl.ds(c * TNP + t * 128, 128)]],
                otrows.at[pl.ds(t * 128, 128), :], sem))
        for cp in cps:
            cp.wait()

        @pl.when(c > 0)
        def _():
            pltpu.make_async_copy(obuf, out_hbm.at[pl.ds(b0 - S, S), :],
                                  osem).wait()

        def row_body(b, carry2):
            x = [drows[b, pl.ds(v * LANES, LANES)] for v in range(VEC // LANES)]
            wb = b * CTX
            for j in range(CTX):
                for v in range(VEC // LANES):
                    x[v] = x[v] + wrows[wb + j, pl.ds(v * LANES, LANES)]
            tb = b * NOISE
            bvec = jnp.full((LANES,), b, jnp.int32)
            for k in range(NOISE):
                p0 = x[0] * otrows[tb + k, pl.ds(0, LANES)]
                p1 = x[1] * otrows[tb + k, pl.ds(LANES, LANES)]
                p2 = x[2] * otrows[tb + k, pl.ds(2 * LANES, LANES)]
                p3 = x[3] * otrows[tb + k, pl.ds(3 * LANES, LANES)]
                cs = plsc.cumsum((p0 + p1) + (p2 + p3))
                plsc.store_scatter(
                    obuf, [bvec, jnp.full((LANES,), k, jnp.int32)], cs,
                    mask=last_lane)
            return carry2

        lax.fori_loop(0, S, row_body, 0)
        pltpu.async_copy(obuf, out_hbm.at[pl.ds(b0, S), :], osem)
        return carry

    lax.fori_loop(0, N_CHUNKS, chunk_body, 0)
    last0 = pl.multiple_of(wid * PER_W + (N_CHUNKS - 1) * S, S)
    pltpu.make_async_copy(obuf, out_hbm.at[pl.ds(last0, S), :], osem).wait()


def kernel(context_ids, doc_ids, target_noise_ids, D, W, O):
    ctx_flat = context_ids.reshape(-1).astype(jnp.int32)
    tn_pad = jnp.pad(
        target_noise_ids.astype(jnp.int32).reshape(-1, S * NOISE),
        ((0, 0), (0, TNP - S * NOISE))).reshape(-1)
    ot = jnp.transpose(O)
    return _dm_sc(ctx_flat, doc_ids.astype(jnp.int32), tn_pad, D, W, ot)


# speedup vs baseline: 6.2419x; 1.8333x over previous
import functools

import jax
import jax.numpy as jnp
from jax import lax
from jax.experimental import pallas as pl
from jax.experimental.pallas import tpu as pltpu
from jax.experimental.pallas import tpu_sc as plsc

VEC = 64
CTX = 20
NOISE = 26
BATCH = 16384
LANES = 16

_info = plsc.get_sparse_core_info()
NC = _info.num_cores
NS = _info.num_subcores
NW = NC * NS
S = 32
PER_W = BATCH // NW
N_CHUNKS = PER_W // S
TNP = 896
CTXC = S * CTX


@functools.partial(
    pl.kernel,
    mesh=plsc.VectorSubcoreMesh(core_axis_name="c", subcore_axis_name="s"),
    compiler_params=pltpu.CompilerParams(
        needs_layout_passes=False, use_tc_tiling_on_sc=False),
    out_type=jax.ShapeDtypeStruct((BATCH, NOISE), jnp.float32),
    scratch_types=[
        pltpu.VMEM((N_CHUNKS * CTXC,), jnp.int32),
        pltpu.VMEM((S,), jnp.int32),
        pltpu.VMEM((N_CHUNKS * TNP,), jnp.int32),
        pltpu.VMEM((CTXC, VEC), jnp.float32),
        pltpu.VMEM((S, VEC), jnp.float32),
        pltpu.VMEM((TNP, VEC), jnp.float32),
        pltpu.VMEM((S, NOISE), jnp.float32),
        pltpu.SemaphoreType.DMA,
        pltpu.SemaphoreType.DMA,
    ],
)
def _dm_sc(ctx_hbm, doc_hbm, tn_hbm, d_hbm, w_hbm, ot_hbm, out_hbm,
           ctx_idx, doc_idx, tn_idx, wrows, drows, otrows, obuf, sem, osem):
    wid = lax.axis_index("s") * NC + lax.axis_index("c")
    last_lane = lax.iota(jnp.int32, LANES) == (LANES - 1)

    wbase0 = pl.multiple_of(wid * (N_CHUNKS * CTXC), N_CHUNKS * CTXC)
    tbase0 = pl.multiple_of(wid * (N_CHUNKS * TNP), N_CHUNKS * TNP)
    pltpu.sync_copy(ctx_hbm.at[pl.ds(wbase0, N_CHUNKS * CTXC)], ctx_idx)
    pltpu.sync_copy(tn_hbm.at[pl.ds(tbase0, N_CHUNKS * TNP)], tn_idx)

    def chunk_body(c, carry):
        b0 = pl.multiple_of(wid * PER_W + c * S, S)

        pltpu.sync_copy(doc_hbm.at[pl.ds(b0, S)], doc_idx)

        cps = []
        for t in range(CTXC // 128):
            cps.append(pltpu.async_copy(
                w_hbm.at[ctx_idx.at[pl.ds(c * CTXC + t * 128, 128)]],
                wrows.at[pl.ds(t * 128, 128), :], sem))
        cps.append(pltpu.async_copy(d_hbm.at[doc_idx], drows, sem))
        for t in range(TNP // 128):
            cps.append(pltpu.async_copy(
                ot_hbm.at[tn_idx.at[pl.ds(c * TNP + t * 128, 128)]],
                otrows.at[pl.ds(t * 128, 128), :], sem))
        for cp in cps:
            cp.wait()

        @pl.when(c > 0)
        def _():
            pltpu.make_async_copy(obuf, out_hbm.at[pl.ds(b0 - S, S), :],
                                  osem).wait()

        def row_body(b, carry2):
            x = [drows[b, pl.ds(v * LANES, LANES)] for v in range(VEC // LANES)]
            wb = b * CTX
            for j in range(CTX):
                for v in range(VEC // LANES):
                    x[v] = x[v] + wrows[wb + j, pl.ds(v * LANES, LANES)]
            tb = b * NOISE
            bvec = jnp.full((LANES,), b, jnp.int32)
            for k in range(NOISE):
                p0 = x[0] * otrows[tb + k, pl.ds(0, LANES)]
                p1 = x[1] * otrows[tb + k, pl.ds(LANES, LANES)]
                p2 = x[2] * otrows[tb + k, pl.ds(2 * LANES, LANES)]
                p3 = x[3] * otrows[tb + k, pl.ds(3 * LANES, LANES)]
                cs = plsc.cumsum((p0 + p1) + (p2 + p3))
                plsc.store_scatter(
                    obuf, [bvec, jnp.full((LANES,), k, jnp.int32)], cs,
                    mask=last_lane)
            return carry2

        lax.fori_loop(0, S, row_body, 0)
        pltpu.async_copy(obuf, out_hbm.at[pl.ds(b0, S), :], osem)
        return carry

    lax.fori_loop(0, N_CHUNKS, chunk_body, 0)
    last0 = pl.multiple_of(wid * PER_W + (N_CHUNKS - 1) * S, S)
    pltpu.make_async_copy(obuf, out_hbm.at[pl.ds(last0, S), :], osem).wait()


def kernel(context_ids, doc_ids, target_noise_ids, D, W, O):
    ctx_flat = context_ids.reshape(-1).astype(jnp.int32)
    tn2 = target_noise_ids.astype(jnp.int32).reshape(-1, S * NOISE)
    nrow = tn2.shape[0]
    pad_vals = (jnp.arange(nrow, dtype=jnp.int32)[:, None] * (TNP - S * NOISE)
                + jnp.arange(TNP - S * NOISE, dtype=jnp.int32)[None, :]) % 100000
    tn_pad = jnp.concatenate([tn2, pad_vals], axis=1).reshape(-1)
    ot = jnp.transpose(O)
    return _dm_sc(ctx_flat, doc_ids.astype(jnp.int32), tn_pad, D, W, ot)
